# Initial kernel scaffold; baseline (speedup 1.0000x reference)
import functools

import jax
import jax.numpy as jnp
from jax import lax
from jax.experimental import pallas as pl
from jax.experimental.pallas import tpu as pltpu
from jax.experimental.pallas import tpu_sc as plsc

N = 10000
E = 320000
CH = 128
R = 8

NC = 2
NS = 16
NW = NC * NS

C = 128
KCH = 80
EPT = KCH * C
E_PAD = NW * EPT
ROWS_PER_TILE = N // NS
ACC_ROWS = N + NS


def _flat_idx_body(t_ref, c_ref, o_ref):
    o_ref[...] = t_ref[...] * N + c_ref[...]


def _transform_body(x_ref, w_ref, o_ref):
    for r in range(R):
        o_ref[r] = jnp.dot(x_ref[...], w_ref[r],
                           preferred_element_type=jnp.float32)


def _combine_body(p_ref, b_ref, o_ref):
    o_ref[...] = p_ref[0] + p_ref[1] + b_ref[...]


def _sc_body(xt_hbm, row_hbm, flat_hbm, z_hbm, out_hbm,
             row_v, flat_v, msg0, msg1, acc, sem0, sem1):
    cid = lax.axis_index("c")
    sid = lax.axis_index("s")
    wid = cid * NS + sid
    base = wid * KCH

    pltpu.sync_copy(row_hbm.at[pl.ds(base, KCH)], row_v)
    pltpu.sync_copy(flat_hbm.at[pl.ds(base, KCH)], flat_v)

    pltpu.sync_copy(z_hbm.at[pl.ds(sid * ROWS_PER_TILE, ROWS_PER_TILE)],
                    acc.at[pl.ds(sid * ROWS_PER_TILE, ROWS_PER_TILE)])
    plsc.subcore_barrier()

    msgs = (msg0, msg1)
    sems = (sem0, sem1)

    pltpu.async_copy(xt_hbm.at[flat_v.at[0]], msg0, sem0)
    pltpu.async_copy(xt_hbm.at[flat_v.at[1]], msg1, sem1)

    def outer(i, carry):
        for b in range(2):
            j = i * 2 + b
            pltpu.make_async_copy(xt_hbm.at[flat_v.at[j]], msgs[b],
                                  sems[b]).wait()
            pltpu.sync_copy(msgs[b], acc.at[row_v.at[j]], add=True)

            @pl.when(j + 2 < KCH)
            def _():
                pltpu.async_copy(xt_hbm.at[flat_v.at[j + 2]], msgs[b], sems[b])
        return carry

    lax.fori_loop(0, KCH // 2, outer, 0)
    plsc.subcore_barrier()

    pltpu.sync_copy(acc.at[pl.ds(sid * ROWS_PER_TILE, ROWS_PER_TILE)],
                    out_hbm.at[cid, pl.ds(sid * ROWS_PER_TILE, ROWS_PER_TILE)])


_sc_gather_scatter = functools.partial(
    pl.kernel,
    mesh=plsc.VectorSubcoreMesh(core_axis_name="c", subcore_axis_name="s"),
    out_type=jax.ShapeDtypeStruct((NC, N, CH), jnp.float32),
    scratch_types=[
        pltpu.VMEM((KCH, C), jnp.int32),
        pltpu.VMEM((KCH, C), jnp.int32),
        pltpu.VMEM((C, CH), jnp.float32),
        pltpu.VMEM((C, CH), jnp.float32),
        pltpu.VMEM_SHARED((ACC_ROWS, CH), jnp.float32),
        pltpu.SemaphoreType.DMA,
        pltpu.SemaphoreType.DMA,
    ],
)(_sc_body)


@jax.jit
def kernel(x, edge_index, edge_type, weight, bias):
    row = edge_index[0]
    col = edge_index[1]
    pad = E_PAD - E
    rowp = jnp.concatenate([row, jnp.full((pad,), N, jnp.int32)])
    colp = jnp.concatenate([col, jnp.zeros((pad,), jnp.int32)])
    typep = jnp.concatenate([edge_type, jnp.zeros((pad,), jnp.int32)])
    row2d = rowp.reshape(E_PAD // C, C)
    col2d = colp.reshape(E_PAD // C, C)
    type2d = typep.reshape(E_PAD // C, C)

    flat2d = pl.pallas_call(
        _flat_idx_body,
        out_shape=jax.ShapeDtypeStruct((E_PAD // C, C), jnp.int32),
    )(type2d, col2d)

    xt = pl.pallas_call(
        _transform_body,
        grid=(10,),
        in_specs=[
            pl.BlockSpec((N // 10, CH), lambda j: (j, 0)),
            pl.BlockSpec((R, CH, CH), lambda j: (0, 0, 0)),
        ],
        out_specs=pl.BlockSpec((R, N // 10, CH), lambda j: (0, j, 0)),
        out_shape=jax.ShapeDtypeStruct((R, N, CH), jnp.float32),
    )(x, weight)
    xt_flat = xt.reshape(R * N, CH)

    zeros = jnp.zeros((N, CH), jnp.float32)
    partials = _sc_gather_scatter(xt_flat, row2d, flat2d, zeros)

    out = pl.pallas_call(
        _combine_body,
        grid=(10,),
        in_specs=[
            pl.BlockSpec((NC, N // 10, CH), lambda j: (0, j, 0)),
            pl.BlockSpec((1, CH), lambda j: (0, 0)),
        ],
        out_specs=pl.BlockSpec((N // 10, CH), lambda j: (j, 0)),
        out_shape=jax.ShapeDtypeStruct((N, CH), jnp.float32),
    )(partials, bias.reshape(1, CH))
    return out


# speedup vs baseline: 9.5025x; 9.5025x over previous
import functools

import jax
import jax.numpy as jnp
from jax import lax
from jax.experimental import pallas as pl
from jax.experimental.pallas import tpu as pltpu
from jax.experimental.pallas import tpu_sc as plsc

N = 10000
E = 320000
CH = 128
R = 8

NC = 2
NS = 16
NW = NC * NS

C = 128
KCH = 80
STAGES = 2
CPS = KCH // STAGES
EPT = KCH * C
E_PAD = NW * EPT
N_ACC = 10240
ROWS_PER_TILE = N_ACC // NS


def _flat_idx_body(t_ref, c_ref, o_ref):
    o_ref[...] = t_ref[...] * N + c_ref[...]


def _transform_body(x_ref, w_ref, o_ref):
    for r in range(R):
        o_ref[r] = jnp.dot(x_ref[...], w_ref[r],
                           preferred_element_type=jnp.float32)


def _combine_body(p_ref, b_ref, o_ref):
    o_ref[...] = p_ref[0] + p_ref[1] + b_ref[...]


def _sc_body(xt_hbm, row_hbm, flat_hbm, z_hbm, out_hbm,
             row_v, flat_v, msg0, msg1, acc, sem0, sem1):
    cid = lax.axis_index("c")
    sid = lax.axis_index("s")
    wid = cid * NS + sid
    base = wid * KCH

    pltpu.sync_copy(z_hbm.at[pl.ds(sid * ROWS_PER_TILE, ROWS_PER_TILE)],
                    acc.at[pl.ds(sid * ROWS_PER_TILE, ROWS_PER_TILE)])
    plsc.subcore_barrier()

    msgs = (msg0, msg1)
    sems = (sem0, sem1)

    for s in range(STAGES):
        pltpu.sync_copy(row_hbm.at[pl.ds(base + s * CPS, CPS)], row_v)
        pltpu.sync_copy(flat_hbm.at[pl.ds(base + s * CPS, CPS)], flat_v)

        pltpu.async_copy(xt_hbm.at[flat_v.at[0]], msg0, sem0)
        pltpu.async_copy(xt_hbm.at[flat_v.at[1]], msg1, sem1)

        def outer(i, carry):
            for b in range(2):
                j = i * 2 + b
                pltpu.make_async_copy(xt_hbm.at[flat_v.at[j]], msgs[b],
                                      sems[b]).wait()
                pltpu.sync_copy(msgs[b], acc.at[row_v.at[j]], add=True)

                @pl.when(j + 2 < CPS)
                def _():
                    pltpu.async_copy(xt_hbm.at[flat_v.at[j + 2]], msgs[b],
                                     sems[b])
            return carry

        lax.fori_loop(0, CPS // 2, outer, 0)
    plsc.subcore_barrier()

    pltpu.sync_copy(acc.at[pl.ds(sid * ROWS_PER_TILE, ROWS_PER_TILE)],
                    out_hbm.at[cid, pl.ds(sid * ROWS_PER_TILE, ROWS_PER_TILE)])


_sc_gather_scatter = functools.partial(
    pl.kernel,
    mesh=plsc.VectorSubcoreMesh(core_axis_name="c", subcore_axis_name="s"),
    out_type=jax.ShapeDtypeStruct((NC, N_ACC, CH), jnp.float32),
    scratch_types=[
        pltpu.VMEM((CPS, C), jnp.int32),
        pltpu.VMEM((CPS, C), jnp.int32),
        pltpu.VMEM((C, CH), jnp.float32),
        pltpu.VMEM((C, CH), jnp.float32),
        pltpu.VMEM_SHARED((N_ACC, CH), jnp.float32),
        pltpu.SemaphoreType.DMA,
        pltpu.SemaphoreType.DMA,
    ],
)(_sc_body)


@jax.jit
def kernel(x, edge_index, edge_type, weight, bias):
    row = edge_index[0]
    col = edge_index[1]
    pad = E_PAD - E
    rowp = jnp.concatenate([row, jnp.full((pad,), N, jnp.int32)])
    colp = jnp.concatenate([col, jnp.zeros((pad,), jnp.int32)])
    typep = jnp.concatenate([edge_type, jnp.zeros((pad,), jnp.int32)])
    row2d = rowp.reshape(E_PAD // C, C)
    col2d = colp.reshape(E_PAD // C, C)
    type2d = typep.reshape(E_PAD // C, C)

    flat2d = pl.pallas_call(
        _flat_idx_body,
        out_shape=jax.ShapeDtypeStruct((E_PAD // C, C), jnp.int32),
    )(type2d, col2d)

    xt = pl.pallas_call(
        _transform_body,
        grid=(10,),
        in_specs=[
            pl.BlockSpec((N // 10, CH), lambda j: (j, 0)),
            pl.BlockSpec((R, CH, CH), lambda j: (0, 0, 0)),
        ],
        out_specs=pl.BlockSpec((R, N // 10, CH), lambda j: (0, j, 0)),
        out_shape=jax.ShapeDtypeStruct((R, N, CH), jnp.float32),
    )(x, weight)
    xt_flat = xt.reshape(R * N, CH)

    zeros = jnp.zeros((N_ACC, CH), jnp.float32)
    partials = _sc_gather_scatter(xt_flat, row2d, flat2d, zeros)

    out = pl.pallas_call(
        _combine_body,
        grid=(10,),
        in_specs=[
            pl.BlockSpec((NC, N // 10, CH), lambda j: (0, j, 0)),
            pl.BlockSpec((1, CH), lambda j: (0, 0)),
        ],
        out_specs=pl.BlockSpec((N // 10, CH), lambda j: (j, 0)),
        out_shape=jax.ShapeDtypeStruct((N, CH), jnp.float32),
    )(partials, bias.reshape(1, CH))
    return out


# speedup vs baseline: 33.1472x; 3.4882x over previous
import functools

import jax
import jax.numpy as jnp
from jax import lax
from jax.experimental import pallas as pl
from jax.experimental.pallas import tpu as pltpu
from jax.experimental.pallas import tpu_sc as plsc

N = 10000
E = 320000
CH = 128
R = 8

NC = 2
NS = 16
NW = NC * NS

C = 128
KCH = 80
STAGES = 2
CPS = KCH // STAGES
EPT = KCH * C
E_PAD = NW * EPT
N_ACC = 10240
ROWS_PER_TILE = N_ACC // NS


def _flat_idx_body(t_ref, c_ref, o_ref):
    o_ref[...] = t_ref[...] * N + c_ref[...]


def _transform_body(x_ref, w_ref, o_ref):
    for r in range(R):
        o_ref[r] = jnp.dot(x_ref[...], w_ref[r],
                           preferred_element_type=jnp.float32)


def _combine_body(p_ref, b_ref, o_ref):
    o_ref[...] = p_ref[0] + p_ref[1] + b_ref[...]


def _sc_body(xt_hbm, row_hbm, flat_hbm, z_hbm, out_hbm,
             row_v, flat_v, msg0, msg1, acc, sem0, sem1):
    cid = lax.axis_index("c")
    sid = lax.axis_index("s")
    wid = cid * NS + sid
    base = wid * KCH

    pltpu.sync_copy(z_hbm.at[pl.ds(sid * ROWS_PER_TILE, ROWS_PER_TILE)],
                    acc.at[pl.ds(sid * ROWS_PER_TILE, ROWS_PER_TILE)])
    plsc.subcore_barrier()

    msgs = (msg0, msg1)
    sems = (sem0, sem1)

    for s in range(STAGES):
        pltpu.sync_copy(row_hbm.at[pl.ds(base + s * CPS, CPS)], row_v)
        pltpu.sync_copy(flat_hbm.at[pl.ds(base + s * CPS, CPS)], flat_v)

        pltpu.async_copy(xt_hbm.at[flat_v.at[0]], msg0, sem0)
        pltpu.async_copy(xt_hbm.at[flat_v.at[1]], msg1, sem1)

        def outer(i, carry):
            for b in range(2):
                j = i * 2 + b
                pltpu.make_async_copy(xt_hbm.at[flat_v.at[j]], msgs[b],
                                      sems[b]).wait()
                pltpu.sync_copy(msgs[b], acc.at[row_v.at[j]], add=True)

                @pl.when(j + 2 < CPS)
                def _():
                    pltpu.async_copy(xt_hbm.at[flat_v.at[j + 2]], msgs[b],
                                     sems[b])
            return carry

        lax.fori_loop(0, CPS // 2, outer, 0)
    plsc.subcore_barrier()

    pltpu.sync_copy(acc.at[pl.ds(sid * ROWS_PER_TILE, ROWS_PER_TILE)],
                    out_hbm.at[cid, pl.ds(sid * ROWS_PER_TILE, ROWS_PER_TILE)])


_sc_gather_scatter = functools.partial(
    pl.kernel,
    mesh=plsc.VectorSubcoreMesh(core_axis_name="c", subcore_axis_name="s"),
    out_type=jax.ShapeDtypeStruct((NC, N_ACC, CH), jnp.float32),
    scratch_types=[
        pltpu.VMEM((CPS, C), jnp.int32),
        pltpu.VMEM((CPS, C), jnp.int32),
        pltpu.VMEM((C, CH), jnp.float32),
        pltpu.VMEM((C, CH), jnp.float32),
        pltpu.VMEM_SHARED((N_ACC, CH), jnp.float32),
        pltpu.SemaphoreType.DMA,
        pltpu.SemaphoreType.DMA,
    ],
)(_sc_body)


@jax.jit
def kernel(x, edge_index, edge_type, weight, bias):
    row = edge_index[0]
    col = edge_index[1]
    pad = E_PAD - E
    spread = jnp.arange(pad, dtype=jnp.int32) % 128
    rowp = jnp.concatenate([row, N + spread])
    colp = jnp.concatenate([col, spread])
    typep = jnp.concatenate([edge_type, jnp.zeros((pad,), jnp.int32)])
    row2d = rowp.reshape(E_PAD // C, C)
    col2d = colp.reshape(E_PAD // C, C)
    type2d = typep.reshape(E_PAD // C, C)

    flat2d = pl.pallas_call(
        _flat_idx_body,
        out_shape=jax.ShapeDtypeStruct((E_PAD // C, C), jnp.int32),
    )(type2d, col2d)

    xt = pl.pallas_call(
        _transform_body,
        grid=(10,),
        in_specs=[
            pl.BlockSpec((N // 10, CH), lambda j: (j, 0)),
            pl.BlockSpec((R, CH, CH), lambda j: (0, 0, 0)),
        ],
        out_specs=pl.BlockSpec((R, N // 10, CH), lambda j: (0, j, 0)),
        out_shape=jax.ShapeDtypeStruct((R, N, CH), jnp.float32),
    )(x, weight)
    xt_flat = xt.reshape(R * N, CH)

    zeros = jnp.zeros((N_ACC, CH), jnp.float32)
    partials = _sc_gather_scatter(xt_flat, row2d, flat2d, zeros)

    out = pl.pallas_call(
        _combine_body,
        grid=(10,),
        in_specs=[
            pl.BlockSpec((NC, N // 10, CH), lambda j: (0, j, 0)),
            pl.BlockSpec((1, CH), lambda j: (0, 0)),
        ],
        out_specs=pl.BlockSpec((N // 10, CH), lambda j: (j, 0)),
        out_shape=jax.ShapeDtypeStruct((N, CH), jnp.float32),
    )(partials, bias.reshape(1, CH))
    return out


# speedup vs baseline: 35.2527x; 1.0635x over previous
import functools

import jax
import jax.numpy as jnp
from jax import lax
from jax.experimental import pallas as pl
from jax.experimental.pallas import tpu as pltpu
from jax.experimental.pallas import tpu_sc as plsc

N = 10000
E = 320000
CH = 128
R = 8

NC = 2
NS = 16
NW = NC * NS

C = 128
KCH = 80
STAGES = 2
CPS = KCH // STAGES
EPT = KCH * C
E_PAD = NW * EPT
N_ACC = 10240
ROWS_PER_TILE = N_ACC // NS


def _prep_body(ei_ref, et_ref, row_ref, flat_ref):
    row_ref[pl.ds(0, E // C)] = ei_ref[0]
    flat_ref[pl.ds(0, E // C)] = et_ref[...] * N + ei_ref[1]
    lane = lax.broadcasted_iota(jnp.int32, ((E_PAD - E) // C, C), 1)
    row_ref[pl.ds(E // C, (E_PAD - E) // C)] = N + lane
    flat_ref[pl.ds(E // C, (E_PAD - E) // C)] = lane


def _transform_body(x_ref, w_ref, o_ref):
    for r in range(R):
        o_ref[r] = jnp.dot(x_ref[...], w_ref[r],
                           preferred_element_type=jnp.float32)


def _combine_body(p_ref, b_ref, o_ref):
    o_ref[...] = p_ref[0] + p_ref[1] + b_ref[...]


def _sc_body(xt_hbm, row_hbm, flat_hbm, z_hbm, out_hbm,
             row_v, flat_v, msg0, msg1, acc, sem0, sem1):
    cid = lax.axis_index("c")
    sid = lax.axis_index("s")
    wid = cid * NS + sid
    base = wid * KCH

    pltpu.sync_copy(z_hbm.at[pl.ds(sid * ROWS_PER_TILE, ROWS_PER_TILE)],
                    acc.at[pl.ds(sid * ROWS_PER_TILE, ROWS_PER_TILE)])
    plsc.subcore_barrier()

    msgs = (msg0, msg1)
    sems = (sem0, sem1)

    for s in range(STAGES):
        pltpu.sync_copy(row_hbm.at[pl.ds(base + s * CPS, CPS)], row_v)
        pltpu.sync_copy(flat_hbm.at[pl.ds(base + s * CPS, CPS)], flat_v)

        pltpu.async_copy(xt_hbm.at[flat_v.at[0]], msg0, sem0)
        pltpu.async_copy(xt_hbm.at[flat_v.at[1]], msg1, sem1)

        def outer(i, carry):
            for b in range(2):
                j = i * 2 + b
                pltpu.make_async_copy(xt_hbm.at[flat_v.at[j]], msgs[b],
                                      sems[b]).wait()
                pltpu.sync_copy(msgs[b], acc.at[row_v.at[j]], add=True)

                @pl.when(j + 2 < CPS)
                def _():
                    pltpu.async_copy(xt_hbm.at[flat_v.at[j + 2]], msgs[b],
                                     sems[b])
            return carry

        lax.fori_loop(0, CPS // 2, outer, 0)
    plsc.subcore_barrier()

    pltpu.sync_copy(acc.at[pl.ds(sid * ROWS_PER_TILE, ROWS_PER_TILE)],
                    out_hbm.at[cid, pl.ds(sid * ROWS_PER_TILE, ROWS_PER_TILE)])


_sc_gather_scatter = functools.partial(
    pl.kernel,
    mesh=plsc.VectorSubcoreMesh(core_axis_name="c", subcore_axis_name="s"),
    out_type=jax.ShapeDtypeStruct((NC, N_ACC, CH), jnp.float32),
    scratch_types=[
        pltpu.VMEM((CPS, C), jnp.int32),
        pltpu.VMEM((CPS, C), jnp.int32),
        pltpu.VMEM((C, CH), jnp.float32),
        pltpu.VMEM((C, CH), jnp.float32),
        pltpu.VMEM_SHARED((N_ACC, CH), jnp.float32),
        pltpu.SemaphoreType.DMA,
        pltpu.SemaphoreType.DMA,
    ],
)(_sc_body)


@jax.jit
def kernel(x, edge_index, edge_type, weight, bias):
    ei3 = edge_index.reshape(2, E // C, C)
    et2 = edge_type.reshape(E // C, C)
    row2d, flat2d = pl.pallas_call(
        _prep_body,
        out_shape=[
            jax.ShapeDtypeStruct((E_PAD // C, C), jnp.int32),
            jax.ShapeDtypeStruct((E_PAD // C, C), jnp.int32),
        ],
    )(ei3, et2)

    xt = pl.pallas_call(
        _transform_body,
        grid=(10,),
        in_specs=[
            pl.BlockSpec((N // 10, CH), lambda j: (j, 0)),
            pl.BlockSpec((R, CH, CH), lambda j: (0, 0, 0)),
        ],
        out_specs=pl.BlockSpec((R, N // 10, CH), lambda j: (0, j, 0)),
        out_shape=jax.ShapeDtypeStruct((R, N, CH), jnp.float32),
    )(x, weight)
    xt_flat = xt.reshape(R * N, CH)

    zeros = jnp.zeros((N_ACC, CH), jnp.float32)
    partials = _sc_gather_scatter(xt_flat, row2d, flat2d, zeros)

    out = pl.pallas_call(
        _combine_body,
        grid=(10,),
        in_specs=[
            pl.BlockSpec((NC, N // 10, CH), lambda j: (0, j, 0)),
            pl.BlockSpec((1, CH), lambda j: (0, 0)),
        ],
        out_specs=pl.BlockSpec((N // 10, CH), lambda j: (j, 0)),
        out_shape=jax.ShapeDtypeStruct((N, CH), jnp.float32),
    )(partials, bias.reshape(1, CH))
    return out
